# baseline (speedup 1.0000x reference)
import functools

import jax
import jax.numpy as jnp
from jax import lax
from jax.experimental import pallas as pl
from jax.experimental.pallas import tpu as pltpu
from jax.experimental.pallas import tpu_sc as plsc

INPUT_SIZE = 1000000
OUTPUT_SIZE = 32
BATCH = 16384
HIST = 50

B = BATCH * HIST
NC = 2
NS = 16
NW = NC * NS
B_PER_W = B // NW
IDX_MINOR = 128
N_IDX_ROWS = B_PER_W // IDX_MINOR
K = 10
CHUNK = K * IDX_MINOR
N_CHUNKS = B_PER_W // CHUNK


def _fire_gathers(table_hbm, idx_v, buf, sem, c):
    for j in range(K):
        pltpu.make_async_copy(
            table_hbm.at[idx_v.at[c * K + j]],
            buf.at[pl.ds(j * IDX_MINOR, IDX_MINOR)],
            sem,
        ).start()


def _wait_gathers(table_hbm, buf, sem):
    pltpu.make_async_copy(table_hbm.at[pl.ds(0, CHUNK)], buf, sem).wait()


def _gather_kernel(table_hbm, idx_hbm, out_hbm, idx_v, buf0, buf1,
                   gsem0, gsem1, wsem0, wsem1):
    wid = lax.axis_index("s") * NC + lax.axis_index("c")
    base = wid * B_PER_W

    pltpu.sync_copy(idx_hbm.at[wid], idx_v)

    def out_at(c):
        return out_hbm.at[pl.ds(base + c * CHUNK, CHUNK)]

    _fire_gathers(table_hbm, idx_v, buf0, gsem0, 0)
    _fire_gathers(table_hbm, idx_v, buf1, gsem1, 1)

    def body(i, carry):
        c = 2 * i
        wb0 = pltpu.make_async_copy(buf0, out_at(c), wsem0)
        wb1 = pltpu.make_async_copy(buf1, out_at(c + 1), wsem1)
        _wait_gathers(table_hbm, buf0, gsem0)
        wb0.start()
        _wait_gathers(table_hbm, buf1, gsem1)
        wb1.start()
        wb0.wait()
        _fire_gathers(table_hbm, idx_v, buf0, gsem0, c + 2)
        wb1.wait()
        _fire_gathers(table_hbm, idx_v, buf1, gsem1, c + 3)
        return carry

    lax.fori_loop(0, N_CHUNKS // 2 - 1, body, 0, unroll=False)

    c = N_CHUNKS - 2
    wb0 = pltpu.make_async_copy(buf0, out_at(c), wsem0)
    wb1 = pltpu.make_async_copy(buf1, out_at(c + 1), wsem1)
    _wait_gathers(table_hbm, buf0, gsem0)
    wb0.start()
    _wait_gathers(table_hbm, buf1, gsem1)
    wb1.start()
    wb0.wait()
    wb1.wait()


@jax.jit
def _embedded(idx_grouped, weights):
    mesh = plsc.VectorSubcoreMesh(core_axis_name="c", subcore_axis_name="s")
    run = functools.partial(
        pl.kernel,
        mesh=mesh,
        out_type=jax.ShapeDtypeStruct((B, OUTPUT_SIZE), jnp.float32),
        scratch_types=[
            pltpu.VMEM((N_IDX_ROWS, IDX_MINOR), jnp.int32),
            pltpu.VMEM((CHUNK, OUTPUT_SIZE), jnp.float32),
            pltpu.VMEM((CHUNK, OUTPUT_SIZE), jnp.float32),
            pltpu.SemaphoreType.DMA,
            pltpu.SemaphoreType.DMA,
            pltpu.SemaphoreType.DMA,
            pltpu.SemaphoreType.DMA,
        ],
        compiler_params=pltpu.CompilerParams(use_tc_tiling_on_sc=False),
    )(_gather_kernel)
    return run(weights, idx_grouped)


def kernel(X, weights):
    idx_grouped = X.reshape(-1).astype(jnp.int32).reshape(NW, N_IDX_ROWS, IDX_MINOR)
    out = _embedded(idx_grouped, weights)
    return out.reshape(BATCH, HIST, OUTPUT_SIZE)


# speedup vs baseline: 1.3902x; 1.3902x over previous
import functools

import jax
import jax.numpy as jnp
from jax import lax
from jax.experimental import pallas as pl
from jax.experimental.pallas import tpu as pltpu
from jax.experimental.pallas import tpu_sc as plsc

INPUT_SIZE = 1000000
OUTPUT_SIZE = 32
BATCH = 16384
HIST = 50

NC = 2
NS = 16
NW = NC * NS
CT_PER_W = 4
HL = 5
N_HC = HIST // HL
LANES = 16


def _gather_kernel(table_hbm, idx_hbm, out_hbm, idx_v, in_buf, stage, gsem, wsem):
    wid = lax.axis_index("s") * NC + lax.axis_index("c")

    pltpu.sync_copy(idx_hbm.at[wid], idx_v)

    iota = lax.iota(jnp.int32, LANES)

    def extract_unit(u):
        u_vec = jnp.full((LANES,), u, jnp.int32)
        for g in range(8):
            rows_g = g * LANES + iota
            for f in range(OUTPUT_SIZE):
                f_vec = jnp.full((LANES,), f, jnp.int32)
                v = plsc.load_gather(in_buf, [u_vec, rows_g, f_vec])
                stage[u, f // 8, f % 8, pl.ds(g * LANES, LANES)] = v

    def group(i, carry):
        ct = i // N_HC
        hc = i - ct * N_HC
        copies = []
        for u in range(HL):
            c = pltpu.make_async_copy(
                table_hbm.at[idx_v.at[ct, hc, u]], in_buf.at[u], gsem)
            c.start()
            copies.append(c)
        for c in copies:
            c.wait()

        @pl.when(i > 0)
        def _():
            pltpu.make_async_copy(
                stage, out_hbm.at[pl.ds(0, HL), :, 0, :, :], wsem).wait()

        def ex_body(u, c2):
            extract_unit(u)
            return c2

        lax.fori_loop(0, HL, ex_body, 0, unroll=False)

        pltpu.make_async_copy(
            stage,
            out_hbm.at[pl.ds(hc * HL, HL), :, wid * CT_PER_W + ct, :, :],
            wsem,
        ).start()
        return carry

    lax.fori_loop(0, CT_PER_W * N_HC, group, 0, unroll=False)
    pltpu.make_async_copy(
        stage, out_hbm.at[pl.ds(0, HL), :, 0, :, :], wsem).wait()


@jax.jit
def _embedded(idx_arranged, weights):
    mesh = plsc.VectorSubcoreMesh(core_axis_name="c", subcore_axis_name="s")
    run = functools.partial(
        pl.kernel,
        mesh=mesh,
        out_type=jax.ShapeDtypeStruct(
            (HIST, OUTPUT_SIZE // 8, BATCH // 128, 8, 128), jnp.float32),
        scratch_types=[
            pltpu.VMEM((CT_PER_W, N_HC, HL, 128), jnp.int32),
            pltpu.VMEM((HL, 128, OUTPUT_SIZE), jnp.float32),
            pltpu.VMEM((HL, OUTPUT_SIZE // 8, 8, 128), jnp.float32),
            pltpu.SemaphoreType.DMA,
            pltpu.SemaphoreType.DMA,
        ],
        compiler_params=pltpu.CompilerParams(
            use_tc_tiling_on_sc=False, needs_layout_passes=False),
    )(_gather_kernel)
    return run(weights, idx_arranged)


def kernel(X, weights):
    idx = X.astype(jnp.int32).reshape(NW, CT_PER_W, 128, N_HC, HL)
    idx = idx.transpose(0, 1, 3, 4, 2)
    phys = _embedded(idx, weights)
    out = phys.transpose(2, 4, 0, 1, 3)
    return out.reshape(BATCH, HIST, OUTPUT_SIZE)


# speedup vs baseline: 1.4254x; 1.0253x over previous
import functools

import jax
import jax.numpy as jnp
from jax import lax
from jax.experimental import pallas as pl
from jax.experimental.pallas import tpu as pltpu
from jax.experimental.pallas import tpu_sc as plsc

INPUT_SIZE = 1000000
OUTPUT_SIZE = 32
BATCH = 16384
HIST = 50

NC = 2
NS = 16
NW = NC * NS
CT_PER_W = 4
HL = 5
N_HC = HIST // HL
N_GROUPS = CT_PER_W * N_HC
LANES = 16


def _gather_kernel(table_hbm, idx_hbm, out_hbm, idx_v, bufa, bufb,
                   stga, stgb, gsa, gsb, wsa, wsb):
    wid = lax.axis_index("s") * NC + lax.axis_index("c")

    pltpu.sync_copy(idx_hbm.at[wid], idx_v)

    iota = lax.iota(jnp.int32, LANES)
    rows = [g * LANES + iota for g in range(8)]

    def fire(i, buf, sem):
        ct = i // N_HC
        hc = i - ct * N_HC
        for u in range(HL):
            pltpu.make_async_copy(
                table_hbm.at[idx_v.at[ct, hc, u]], buf.at[u], sem).start()

    def wait_g(buf, sem):
        pltpu.make_async_copy(table_hbm.at[pl.ds(0, HL * 128)],
                              buf, sem).wait()

    def wait_w(stg, sem):
        pltpu.make_async_copy(stg, out_hbm.at[pl.ds(0, HL), :, 0, :, :],
                              sem).wait()

    def extract(buf, stg):
        def unit(u, carry):
            u_vec = jnp.full((LANES,), u, jnp.int32)
            for f in range(OUTPUT_SIZE):
                f_vec = jnp.full((LANES,), f, jnp.int32)
                for g in range(8):
                    v = plsc.load_gather(buf, [u_vec, rows[g], f_vec])
                    stg[u, f // 8, f % 8, pl.ds(g * LANES, LANES)] = v
            return carry
        lax.fori_loop(0, HL, unit, 0, unroll=False)

    def put(i, stg, sem):
        ct = i // N_HC
        hc = i - ct * N_HC
        pltpu.make_async_copy(
            stg,
            out_hbm.at[pl.ds(hc * HL, HL), :, wid * CT_PER_W + ct, :, :],
            sem,
        ).start()

    fire(0, bufa, gsa)
    fire(1, bufb, gsb)

    def body(i, carry):
        g0 = 2 * i
        wait_g(bufa, gsa)

        @pl.when(i > 0)
        def _():
            wait_w(stga, wsa)

        extract(bufa, stga)

        @pl.when(g0 + 2 < N_GROUPS)
        def _():
            fire(g0 + 2, bufa, gsa)

        put(g0, stga, wsa)

        wait_g(bufb, gsb)

        @pl.when(i > 0)
        def _():
            wait_w(stgb, wsb)

        extract(bufb, stgb)

        @pl.when(g0 + 3 < N_GROUPS)
        def _():
            fire(g0 + 3, bufb, gsb)

        put(g0 + 1, stgb, wsb)
        return carry

    lax.fori_loop(0, N_GROUPS // 2, body, 0, unroll=False)
    wait_w(stga, wsa)
    wait_w(stgb, wsb)


@jax.jit
def _embedded(idx_arranged, weights):
    mesh = plsc.VectorSubcoreMesh(core_axis_name="c", subcore_axis_name="s")
    run = functools.partial(
        pl.kernel,
        mesh=mesh,
        out_type=jax.ShapeDtypeStruct(
            (HIST, OUTPUT_SIZE // 8, BATCH // 128, 8, 128), jnp.float32),
        scratch_types=[
            pltpu.VMEM((CT_PER_W, N_HC, HL, 128), jnp.int32),
            pltpu.VMEM((HL, 128, OUTPUT_SIZE), jnp.float32),
            pltpu.VMEM((HL, 128, OUTPUT_SIZE), jnp.float32),
            pltpu.VMEM((HL, OUTPUT_SIZE // 8, 8, 128), jnp.float32),
            pltpu.VMEM((HL, OUTPUT_SIZE // 8, 8, 128), jnp.float32),
            pltpu.SemaphoreType.DMA,
            pltpu.SemaphoreType.DMA,
            pltpu.SemaphoreType.DMA,
            pltpu.SemaphoreType.DMA,
        ],
        compiler_params=pltpu.CompilerParams(
            use_tc_tiling_on_sc=False, needs_layout_passes=False),
    )(_gather_kernel)
    return run(weights, idx_arranged)


def kernel(X, weights):
    idx = X.astype(jnp.int32).reshape(NW, CT_PER_W, 128, N_HC, HL)
    idx = idx.transpose(0, 1, 3, 4, 2)
    phys = _embedded(idx, weights)
    out = phys.transpose(2, 4, 0, 1, 3)
    return out.reshape(BATCH, HIST, OUTPUT_SIZE)


# speedup vs baseline: 1.4811x; 1.0391x over previous
import functools

import jax
import jax.numpy as jnp
from jax import lax
from jax.experimental import pallas as pl
from jax.experimental.pallas import tpu as pltpu
from jax.experimental.pallas import tpu_sc as plsc

INPUT_SIZE = 1000000
OUTPUT_SIZE = 32
BATCH = 16384
HIST = 50

NC = 2
NS = 16
NW = NC * NS
CT_PER_W = 4
HL = 5
N_HC = HIST // HL
N_GROUPS = CT_PER_W * N_HC
LANES = 16


def _gather_kernel(table_hbm, idx_hbm, out_hbm, idx_v, bufa, bufb,
                   stga, stgb, gsa, gsb, wsa, wsb):
    wid = lax.axis_index("s") * NC + lax.axis_index("c")

    pltpu.sync_copy(idx_hbm.at[wid], idx_v)

    iota = lax.iota(jnp.int32, LANES)
    rows = [g * LANES + iota for g in range(8)]
    zeros = jnp.zeros((LANES,), jnp.int32)
    ones = jnp.full((LANES,), 1, jnp.int32)

    def fire(i, buf, sem):
        ct = i // N_HC
        hc = i - ct * N_HC
        for u in range(HL):
            pltpu.make_async_copy(
                table_hbm.at[idx_v.at[ct, hc, u]],
                buf.at[pl.ds(u * 128, 128), :], sem).start()

    def wait_g(buf, sem):
        pltpu.make_async_copy(table_hbm.at[pl.ds(0, HL * 128)],
                              buf, sem).wait()

    def wait_w(stg, sem):
        pltpu.make_async_copy(stg, out_hbm.at[pl.ds(0, HL), :, 0, :, :],
                              sem).wait()

    def extract(buf, stg):
        def unit(u, carry):
            ub = u * 128
            urows = [ub + rows[g] for g in range(8)]
            f_vec = zeros
            for f in range(OUTPUT_SIZE):
                for g in range(8):
                    v = plsc.load_gather(buf, [urows[g], f_vec])
                    stg[u, f // 8, f % 8, pl.ds(g * LANES, LANES)] = v
                f_vec = f_vec + ones
            return carry
        lax.fori_loop(0, HL, unit, 0, unroll=False)

    def put(i, stg, sem):
        ct = i // N_HC
        hc = i - ct * N_HC
        pltpu.make_async_copy(
            stg,
            out_hbm.at[pl.ds(hc * HL, HL), :, wid * CT_PER_W + ct, :, :],
            sem,
        ).start()

    fire(0, bufa, gsa)
    fire(1, bufb, gsb)

    def body(i, carry):
        g0 = 2 * i
        wait_g(bufa, gsa)

        @pl.when(i > 0)
        def _():
            wait_w(stga, wsa)

        extract(bufa, stga)

        @pl.when(g0 + 2 < N_GROUPS)
        def _():
            fire(g0 + 2, bufa, gsa)

        put(g0, stga, wsa)

        wait_g(bufb, gsb)

        @pl.when(i > 0)
        def _():
            wait_w(stgb, wsb)

        extract(bufb, stgb)

        @pl.when(g0 + 3 < N_GROUPS)
        def _():
            fire(g0 + 3, bufb, gsb)

        put(g0 + 1, stgb, wsb)
        return carry

    lax.fori_loop(0, N_GROUPS // 2, body, 0, unroll=False)
    wait_w(stga, wsa)
    wait_w(stgb, wsb)


@jax.jit
def _embedded(idx_arranged, weights):
    mesh = plsc.VectorSubcoreMesh(core_axis_name="c", subcore_axis_name="s")
    run = functools.partial(
        pl.kernel,
        mesh=mesh,
        out_type=jax.ShapeDtypeStruct(
            (HIST, OUTPUT_SIZE // 8, BATCH // 128, 8, 128), jnp.float32),
        scratch_types=[
            pltpu.VMEM((CT_PER_W, N_HC, HL, 128), jnp.int32),
            pltpu.VMEM((HL * 128, OUTPUT_SIZE), jnp.float32),
            pltpu.VMEM((HL * 128, OUTPUT_SIZE), jnp.float32),
            pltpu.VMEM((HL, OUTPUT_SIZE // 8, 8, 128), jnp.float32),
            pltpu.VMEM((HL, OUTPUT_SIZE // 8, 8, 128), jnp.float32),
            pltpu.SemaphoreType.DMA,
            pltpu.SemaphoreType.DMA,
            pltpu.SemaphoreType.DMA,
            pltpu.SemaphoreType.DMA,
        ],
        compiler_params=pltpu.CompilerParams(
            use_tc_tiling_on_sc=False, needs_layout_passes=False),
    )(_gather_kernel)
    return run(weights, idx_arranged)


def kernel(X, weights):
    idx = X.astype(jnp.int32).reshape(NW, CT_PER_W, 128, N_HC, HL)
    idx = idx.transpose(0, 1, 3, 4, 2)
    phys = _embedded(idx, weights)
    out = phys.transpose(2, 4, 0, 1, 3)
    return out.reshape(BATCH, HIST, OUTPUT_SIZE)


# speedup vs baseline: 1.9951x; 1.3471x over previous
import functools

import jax
import jax.numpy as jnp
from jax import lax
from jax.experimental import pallas as pl
from jax.experimental.pallas import tpu as pltpu
from jax.experimental.pallas import tpu_sc as plsc

INPUT_SIZE = 1000000
OUTPUT_SIZE = 32
BATCH = 16384
HIST = 50

NC = 2
NS = 16
NW = NC * NS
CT_PER_W = 4
HL = 5
N_HC = HIST // HL
N_GROUPS = CT_PER_W * N_HC
LANES = 16


def _gather_kernel(table_hbm, idx_hbm, out_hbm, idx_v, bufa, bufb,
                   stga, stgb, gsa, gsb, wsa, wsb):
    wid = lax.axis_index("s") * NC + lax.axis_index("c")

    pltpu.sync_copy(idx_hbm.at[wid], idx_v)

    iota = lax.iota(jnp.int32, LANES)
    rows = [g * LANES + iota for g in range(8)]
    zeros = jnp.zeros((LANES,), jnp.int32)
    ones = jnp.full((LANES,), 1, jnp.int32)

    def fire(i, buf, sem):
        ct = i // N_HC
        hc = i - ct * N_HC
        for u in range(HL):
            pltpu.make_async_copy(
                table_hbm.at[idx_v.at[ct, hc, u]],
                buf.at[pl.ds(u * 128, 128), :], sem).start()

    def wait_g(buf, sem):
        pltpu.make_async_copy(table_hbm.at[pl.ds(0, HL * 128)],
                              buf, sem).wait()

    def wait_w(stg, sem):
        pltpu.make_async_copy(stg, out_hbm.at[pl.ds(0, HL), :, 0, :, :],
                              sem).wait()

    def extract(buf, stg):
        def unit(u, carry):
            ub = u * 128
            urows = [ub + rows[g] for g in range(8)]

            @plsc.parallel_loop(0, OUTPUT_SIZE, 1, unroll=2)
            def fbody(f):
                f_vec = zeros + f
                for g in range(8):
                    v = plsc.load_gather(buf, [urows[g], f_vec])
                    stg[u, f // 8, f % 8, pl.ds(g * LANES, LANES)] = v

            return carry
        lax.fori_loop(0, HL, unit, 0, unroll=False)

    def put(i, stg, sem):
        ct = i // N_HC
        hc = i - ct * N_HC
        pltpu.make_async_copy(
            stg,
            out_hbm.at[pl.ds(hc * HL, HL), :, wid * CT_PER_W + ct, :, :],
            sem,
        ).start()

    fire(0, bufa, gsa)
    fire(1, bufb, gsb)

    def body(i, carry):
        g0 = 2 * i
        wait_g(bufa, gsa)

        @pl.when(i > 0)
        def _():
            wait_w(stga, wsa)

        extract(bufa, stga)

        @pl.when(g0 + 2 < N_GROUPS)
        def _():
            fire(g0 + 2, bufa, gsa)

        put(g0, stga, wsa)

        wait_g(bufb, gsb)

        @pl.when(i > 0)
        def _():
            wait_w(stgb, wsb)

        extract(bufb, stgb)

        @pl.when(g0 + 3 < N_GROUPS)
        def _():
            fire(g0 + 3, bufb, gsb)

        put(g0 + 1, stgb, wsb)
        return carry

    lax.fori_loop(0, N_GROUPS // 2, body, 0, unroll=False)
    wait_w(stga, wsa)
    wait_w(stgb, wsb)


@jax.jit
def _embedded(idx_arranged, weights):
    mesh = plsc.VectorSubcoreMesh(core_axis_name="c", subcore_axis_name="s")
    run = functools.partial(
        pl.kernel,
        mesh=mesh,
        out_type=jax.ShapeDtypeStruct(
            (HIST, OUTPUT_SIZE // 8, BATCH // 128, 8, 128), jnp.float32),
        scratch_types=[
            pltpu.VMEM((CT_PER_W, N_HC, HL, 128), jnp.int32),
            pltpu.VMEM((HL * 128, OUTPUT_SIZE), jnp.float32),
            pltpu.VMEM((HL * 128, OUTPUT_SIZE), jnp.float32),
            pltpu.VMEM((HL, OUTPUT_SIZE // 8, 8, 128), jnp.float32),
            pltpu.VMEM((HL, OUTPUT_SIZE // 8, 8, 128), jnp.float32),
            pltpu.SemaphoreType.DMA,
            pltpu.SemaphoreType.DMA,
            pltpu.SemaphoreType.DMA,
            pltpu.SemaphoreType.DMA,
        ],
        compiler_params=pltpu.CompilerParams(
            use_tc_tiling_on_sc=False, needs_layout_passes=False),
    )(_gather_kernel)
    return run(weights, idx_arranged)


def kernel(X, weights):
    idx = X.astype(jnp.int32).reshape(NW, CT_PER_W, 128, N_HC, HL)
    idx = idx.transpose(0, 1, 3, 4, 2)
    phys = _embedded(idx, weights)
    out = phys.transpose(2, 4, 0, 1, 3)
    return out.reshape(BATCH, HIST, OUTPUT_SIZE)


# speedup vs baseline: 3.0657x; 1.5366x over previous
import functools

import jax
import jax.numpy as jnp
from jax import lax
from jax.experimental import pallas as pl
from jax.experimental.pallas import tpu as pltpu
from jax.experimental.pallas import tpu_sc as plsc

INPUT_SIZE = 1000000
OUTPUT_SIZE = 32
BATCH = 16384
HIST = 50

NC = 2
NS = 16
NW = NC * NS
CT_PER_W = 4
HL = 5
N_HC = HIST // HL
N_GROUPS = CT_PER_W * N_HC
LANES = 16


def _gather_kernel(table_hbm, idx_hbm, out_hbm, idx_v, bufa, bufb,
                   stga, stgb, gsa, gsb, wsa, wsb):
    wid = lax.axis_index("s") * NC + lax.axis_index("c")

    pltpu.sync_copy(idx_hbm.at[wid], idx_v)

    iota = lax.iota(jnp.int32, LANES)
    rows = [g * LANES + iota for g in range(8)]
    zeros = jnp.zeros((LANES,), jnp.int32)

    def fire(i, buf, sem):
        ct = i // N_HC
        hc = i - ct * N_HC
        for u in range(HL):
            pltpu.make_async_copy(
                table_hbm.at[idx_v.at[ct, hc, u]],
                buf.at[pl.ds(u * 128, 128), :], sem).start()

    def wait_g(buf, sem):
        pltpu.make_async_copy(table_hbm.at[pl.ds(0, HL * 128)],
                              buf, sem).wait()

    def wait_w(stg, sem):
        pltpu.make_async_copy(stg, out_hbm.at[pl.ds(0, HL), :, 0, :, :],
                              sem).wait()

    def extract(buf, stg):
        def unit(u, carry):
            ub = u * 128
            urows = [ub + rows[g] for g in range(8)]
            u_vec = zeros + u

            @plsc.parallel_loop(0, OUTPUT_SIZE, 1, unroll=4)
            def fbody(f):
                rot = (iota + f) & (OUTPUT_SIZE - 1)
                rhi = rot >> 3
                rlo = rot & 7
                for g in range(8):
                    v = plsc.load_gather(buf, [urows[g], rot])
                    plsc.store_scatter(stg, [u_vec, rhi, rlo, rows[g]], v)

            return carry
        lax.fori_loop(0, HL, unit, 0, unroll=False)

    def put(i, stg, sem):
        ct = i // N_HC
        hc = i - ct * N_HC
        pltpu.make_async_copy(
            stg,
            out_hbm.at[pl.ds(hc * HL, HL), :, wid * CT_PER_W + ct, :, :],
            sem,
        ).start()

    fire(0, bufa, gsa)
    fire(1, bufb, gsb)

    def body(i, carry):
        g0 = 2 * i
        wait_g(bufa, gsa)

        @pl.when(i > 0)
        def _():
            wait_w(stga, wsa)

        extract(bufa, stga)

        @pl.when(g0 + 2 < N_GROUPS)
        def _():
            fire(g0 + 2, bufa, gsa)

        put(g0, stga, wsa)

        wait_g(bufb, gsb)

        @pl.when(i > 0)
        def _():
            wait_w(stgb, wsb)

        extract(bufb, stgb)

        @pl.when(g0 + 3 < N_GROUPS)
        def _():
            fire(g0 + 3, bufb, gsb)

        put(g0 + 1, stgb, wsb)
        return carry

    lax.fori_loop(0, N_GROUPS // 2, body, 0, unroll=False)
    wait_w(stga, wsa)
    wait_w(stgb, wsb)


N_FULL_TILES = INPUT_SIZE // 128
TAIL_COLS = INPUT_SIZE - N_FULL_TILES * 128


def _transpose_kernel(wt_hbm, out_hbm, ina, inb, sta, stb, isa, isb, osa, osb):
    wid = lax.axis_index("s") * NC + lax.axis_index("c")

    iota = lax.iota(jnp.int32, LANES)
    rows_hi = iota + LANES
    zeros = jnp.zeros((LANES,), jnp.int32)

    def fire_in(t, buf, sem):
        pltpu.make_async_copy(
            wt_hbm.at[:, pl.ds(t * 128, 128)], buf, sem).start()

    def wait_in(buf, sem):
        pltpu.make_async_copy(wt_hbm.at[:, pl.ds(0, 128)], buf, sem).wait()

    def transpose(buf, stg, ncols):
        @plsc.parallel_loop(0, ncols // 4, 1, unroll=2)
        def cbody(c4):
            base = zeros + c4 * 4
            for dc in range(4):
                cv = base + dc
                v0 = plsc.load_gather(buf, [iota, cv])
                v1 = plsc.load_gather(buf, [rows_hi, cv])
                stg[c4, pl.ds(dc * 32, 16)] = v0
                stg[c4, pl.ds(dc * 32 + 16, 16)] = v1

    def fire_out(t, stg, sem):
        pltpu.make_async_copy(
            stg, out_hbm.at[pl.ds(t * 32, 32), :], sem).start()

    def wait_out(stg, sem):
        pltpu.make_async_copy(stg, out_hbm.at[pl.ds(0, 32), :], sem).wait()

    fire_in(wid, ina, isa)
    fire_in(NW + wid, inb, isb)

    def body(i, carry):
        ta = (2 * i) * NW + wid
        tb = ta + NW

        @pl.when(ta < N_FULL_TILES)
        def _():
            @pl.when(i > 0)
            def _():
                wait_out(sta, osa)
            wait_in(ina, isa)
            transpose(ina, sta, 128)

            @pl.when(ta + 2 * NW < N_FULL_TILES)
            def _():
                fire_in(ta + 2 * NW, ina, isa)
            fire_out(ta, sta, osa)

        @pl.when(tb < N_FULL_TILES)
        def _():
            @pl.when(i > 0)
            def _():
                wait_out(stb, osb)
            wait_in(inb, isb)
            transpose(inb, stb, 128)

            @pl.when(tb + 2 * NW < N_FULL_TILES)
            def _():
                fire_in(tb + 2 * NW, inb, isb)
            fire_out(tb, stb, osb)

        return carry

    n_pairs = N_FULL_TILES // (2 * NW) + 1
    lax.fori_loop(0, n_pairs, body, 0, unroll=False)
    wait_out(sta, osa)
    wait_out(stb, osb)


@jax.jit
def _format_table(weights):
    wt = weights.T
    mesh = plsc.VectorSubcoreMesh(core_axis_name="c", subcore_axis_name="s")
    run = functools.partial(
        pl.kernel,
        mesh=mesh,
        out_type=jax.ShapeDtypeStruct((INPUT_SIZE // 4, 128), jnp.float32),
        scratch_types=[
            pltpu.VMEM((OUTPUT_SIZE, 128), jnp.float32),
            pltpu.VMEM((OUTPUT_SIZE, 128), jnp.float32),
            pltpu.VMEM((OUTPUT_SIZE, 128), jnp.float32),
            pltpu.VMEM((OUTPUT_SIZE, 128), jnp.float32),
            pltpu.SemaphoreType.DMA,
            pltpu.SemaphoreType.DMA,
            pltpu.SemaphoreType.DMA,
            pltpu.SemaphoreType.DMA,
        ],
        compiler_params=pltpu.CompilerParams(
            use_tc_tiling_on_sc=True, needs_layout_passes=False),
    )(_transpose_kernel)
    raw = run(wt)
    tail = weights[N_FULL_TILES * 128:, :].reshape(TAIL_COLS // 4, 128)
    raw = lax.dynamic_update_slice(raw, tail, (N_FULL_TILES * 32, 0))
    return raw.reshape(INPUT_SIZE, OUTPUT_SIZE)


@jax.jit
def _embedded(idx_arranged, weights):
    mesh = plsc.VectorSubcoreMesh(core_axis_name="c", subcore_axis_name="s")
    run = functools.partial(
        pl.kernel,
        mesh=mesh,
        out_type=jax.ShapeDtypeStruct(
            (HIST, OUTPUT_SIZE // 8, BATCH // 128, 8, 128), jnp.float32),
        scratch_types=[
            pltpu.VMEM((CT_PER_W, N_HC, HL, 128), jnp.int32),
            pltpu.VMEM((HL * 128, OUTPUT_SIZE), jnp.float32),
            pltpu.VMEM((HL * 128, OUTPUT_SIZE), jnp.float32),
            pltpu.VMEM((HL, OUTPUT_SIZE // 8, 8, 128), jnp.float32),
            pltpu.VMEM((HL, OUTPUT_SIZE // 8, 8, 128), jnp.float32),
            pltpu.SemaphoreType.DMA,
            pltpu.SemaphoreType.DMA,
            pltpu.SemaphoreType.DMA,
            pltpu.SemaphoreType.DMA,
        ],
        compiler_params=pltpu.CompilerParams(
            use_tc_tiling_on_sc=False, needs_layout_passes=False),
    )(_gather_kernel)
    return run(weights, idx_arranged)


def kernel(X, weights):
    idx = X.astype(jnp.int32).reshape(NW, CT_PER_W, 128, N_HC, HL)
    idx = idx.transpose(0, 1, 3, 4, 2)
    tbl = _format_table(weights)
    phys = _embedded(idx, tbl)
    out = phys.transpose(2, 4, 0, 1, 3)
    return out.reshape(BATCH, HIST, OUTPUT_SIZE)


# speedup vs baseline: 6.6804x; 2.1791x over previous
import functools

import jax
import jax.numpy as jnp
from jax import lax
from jax.experimental import pallas as pl
from jax.experimental.pallas import tpu as pltpu
from jax.experimental.pallas import tpu_sc as plsc

INPUT_SIZE = 1000000
OUTPUT_SIZE = 32
BATCH = 16384
HIST = 50

NC = 2
NS = 16
NW = NC * NS
CT_PER_W = 4
HL = 5
N_HC = HIST // HL
N_GROUPS = CT_PER_W * N_HC
LANES = 16


def _gather_kernel(table_hbm, idx_hbm, out_hbm, idx_v, bufa, bufb,
                   stga, stgb, gsa, gsb, wsa, wsb):
    wid = lax.axis_index("s") * NC + lax.axis_index("c")

    pltpu.sync_copy(idx_hbm.at[wid], idx_v)

    iota = lax.iota(jnp.int32, LANES)
    rows = [g * LANES + iota for g in range(8)]
    zeros = jnp.zeros((LANES,), jnp.int32)

    def fire(i, buf, sem):
        ct = i // N_HC
        hc = i - ct * N_HC
        for u in range(HL):
            pltpu.make_async_copy(
                table_hbm.at[idx_v.at[ct, hc, u]],
                buf.at[pl.ds(u * 128, 128), :], sem).start()

    def wait_g(buf, sem):
        pltpu.make_async_copy(table_hbm.at[pl.ds(0, HL * 128)],
                              buf, sem).wait()

    def wait_w(stg, sem):
        pltpu.make_async_copy(stg, out_hbm.at[pl.ds(0, HL), :, 0, :, :],
                              sem).wait()

    def extract(buf, stg):
        def unit(u, carry):
            ub = u * 128
            urows = [ub + rows[g] for g in range(8)]
            u_vec = zeros + u

            @plsc.parallel_loop(0, OUTPUT_SIZE, 1, unroll=4)
            def fbody(f):
                rot = (iota + f) & (OUTPUT_SIZE - 1)
                rhi = rot >> 3
                rlo = rot & 7
                for g in range(8):
                    v = plsc.load_gather(buf, [urows[g], rot])
                    plsc.store_scatter(stg, [u_vec, rhi, rlo, rows[g]], v)

            return carry
        lax.fori_loop(0, HL, unit, 0, unroll=False)

    def put(i, stg, sem):
        ct = i // N_HC
        hc = i - ct * N_HC
        pltpu.make_async_copy(
            stg,
            out_hbm.at[pl.ds(hc * HL, HL), :, wid * CT_PER_W + ct, :, :],
            sem,
        ).start()

    fire(0, bufa, gsa)
    fire(1, bufb, gsb)

    def body(i, carry):
        g0 = 2 * i
        wait_g(bufa, gsa)

        @pl.when(i > 0)
        def _():
            wait_w(stga, wsa)

        extract(bufa, stga)

        @pl.when(g0 + 2 < N_GROUPS)
        def _():
            fire(g0 + 2, bufa, gsa)

        put(g0, stga, wsa)

        wait_g(bufb, gsb)

        @pl.when(i > 0)
        def _():
            wait_w(stgb, wsb)

        extract(bufb, stgb)

        @pl.when(g0 + 3 < N_GROUPS)
        def _():
            fire(g0 + 3, bufb, gsb)

        put(g0 + 1, stgb, wsb)
        return carry

    lax.fori_loop(0, N_GROUPS // 2, body, 0, unroll=False)
    wait_w(stga, wsa)
    wait_w(stgb, wsb)


N_FULL_TILES = INPUT_SIZE // 128
TAIL_COLS = INPUT_SIZE - N_FULL_TILES * 128


def _transpose_kernel(wt_hbm, out_hbm, ina, inb, sta, stb, isa, isb, osa, osb):
    wid = lax.axis_index("s") * NC + lax.axis_index("c")

    iota = lax.iota(jnp.int32, LANES)
    zeros = jnp.zeros((LANES,), jnp.int32)
    cvecs = [c16 * LANES + iota for c16 in range(8)]
    crows = [c16 * 4 + (iota >> 2) for c16 in range(8)]
    ccol_base = (iota & 3) * 32

    def fire_in(t, buf, sem):
        pltpu.make_async_copy(
            wt_hbm.at[:, pl.ds(t * 128, 128)], buf, sem).start()

    def wait_in(buf, sem):
        pltpu.make_async_copy(wt_hbm.at[:, pl.ds(0, 128)], buf, sem).wait()

    def transpose(buf, stg, ncols):
        del ncols

        @plsc.parallel_loop(0, OUTPUT_SIZE, 1, unroll=2)
        def rbody(r):
            rrv = (zeros + r + iota) & (OUTPUT_SIZE - 1)
            for c16 in range(8):
                v = plsc.load_gather(buf, [rrv, cvecs[c16]])
                plsc.store_scatter(stg, [crows[c16], ccol_base + rrv], v)

    def fire_out(t, stg, sem):
        pltpu.make_async_copy(
            stg, out_hbm.at[pl.ds(t * 32, 32), :], sem).start()

    def wait_out(stg, sem):
        pltpu.make_async_copy(stg, out_hbm.at[pl.ds(0, 32), :], sem).wait()

    fire_in(wid, ina, isa)
    fire_in(NW + wid, inb, isb)

    def body(i, carry):
        ta = (2 * i) * NW + wid
        tb = ta + NW

        @pl.when(ta < N_FULL_TILES)
        def _():
            @pl.when(i > 0)
            def _():
                wait_out(sta, osa)
            wait_in(ina, isa)
            transpose(ina, sta, 128)

            @pl.when(ta + 2 * NW < N_FULL_TILES)
            def _():
                fire_in(ta + 2 * NW, ina, isa)
            fire_out(ta, sta, osa)

        @pl.when(tb < N_FULL_TILES)
        def _():
            @pl.when(i > 0)
            def _():
                wait_out(stb, osb)
            wait_in(inb, isb)
            transpose(inb, stb, 128)

            @pl.when(tb + 2 * NW < N_FULL_TILES)
            def _():
                fire_in(tb + 2 * NW, inb, isb)
            fire_out(tb, stb, osb)

        return carry

    n_pairs = N_FULL_TILES // (2 * NW) + 1
    lax.fori_loop(0, n_pairs, body, 0, unroll=False)
    wait_out(sta, osa)
    wait_out(stb, osb)


@jax.jit
def _format_table(weights):
    wt = weights.T
    mesh = plsc.VectorSubcoreMesh(core_axis_name="c", subcore_axis_name="s")
    run = functools.partial(
        pl.kernel,
        mesh=mesh,
        out_type=jax.ShapeDtypeStruct((INPUT_SIZE // 4, 128), jnp.float32),
        scratch_types=[
            pltpu.VMEM((OUTPUT_SIZE, 128), jnp.float32),
            pltpu.VMEM((OUTPUT_SIZE, 128), jnp.float32),
            pltpu.VMEM((OUTPUT_SIZE, 128), jnp.float32),
            pltpu.VMEM((OUTPUT_SIZE, 128), jnp.float32),
            pltpu.SemaphoreType.DMA,
            pltpu.SemaphoreType.DMA,
            pltpu.SemaphoreType.DMA,
            pltpu.SemaphoreType.DMA,
        ],
        compiler_params=pltpu.CompilerParams(
            use_tc_tiling_on_sc=True, needs_layout_passes=False),
    )(_transpose_kernel)
    raw = run(wt)
    tail = weights[N_FULL_TILES * 128:, :].reshape(TAIL_COLS // 4, 128)
    raw = lax.dynamic_update_slice(raw, tail, (N_FULL_TILES * 32, 0))
    return raw.reshape(INPUT_SIZE, OUTPUT_SIZE)


@jax.jit
def _embedded(idx_arranged, weights):
    mesh = plsc.VectorSubcoreMesh(core_axis_name="c", subcore_axis_name="s")
    run = functools.partial(
        pl.kernel,
        mesh=mesh,
        out_type=jax.ShapeDtypeStruct(
            (HIST, OUTPUT_SIZE // 8, BATCH // 128, 8, 128), jnp.float32),
        scratch_types=[
            pltpu.VMEM((CT_PER_W, N_HC, HL, 128), jnp.int32),
            pltpu.VMEM((HL * 128, OUTPUT_SIZE), jnp.float32),
            pltpu.VMEM((HL * 128, OUTPUT_SIZE), jnp.float32),
            pltpu.VMEM((HL, OUTPUT_SIZE // 8, 8, 128), jnp.float32),
            pltpu.VMEM((HL, OUTPUT_SIZE // 8, 8, 128), jnp.float32),
            pltpu.SemaphoreType.DMA,
            pltpu.SemaphoreType.DMA,
            pltpu.SemaphoreType.DMA,
            pltpu.SemaphoreType.DMA,
        ],
        compiler_params=pltpu.CompilerParams(
            use_tc_tiling_on_sc=False, needs_layout_passes=False),
    )(_gather_kernel)
    return run(weights, idx_arranged)


def kernel(X, weights):
    idx = X.astype(jnp.int32).reshape(NW, CT_PER_W, 128, N_HC, HL)
    idx = idx.transpose(0, 1, 3, 4, 2)
    tbl = _format_table(weights)
    phys = _embedded(idx, tbl)
    out = phys.transpose(2, 4, 0, 1, 3)
    return out.reshape(BATCH, HIST, OUTPUT_SIZE)
